# baseline (speedup 1.0000x reference)
import functools

import jax
import jax.numpy as jnp
from jax import lax
from jax.experimental import pallas as pl
from jax.experimental.pallas import tpu as pltpu
from jax.experimental.pallas import tpu_sc as plsc

N_TO = 16384
N_FROM = 4096
KNN = 3
D_FEAT = 512
D_TO = 256
EPS = 1e-5


_B_KNN = 256
_EXP_OFF = 64 << 23


def _knn_body(to_ref, from_aug_ref, idx_ref, w_ref):
    d = jnp.dot(to_ref[...], from_aug_ref[...],
                preferred_element_type=jnp.float32,
                precision=lax.Precision.HIGHEST)
    d = jnp.maximum(d, 0.0)
    dbits = lax.bitcast_convert_type(d, jnp.int32)
    INF = jnp.float32(jnp.inf)
    B = d.shape[0]
    m1 = jnp.full((B, 128), INF, jnp.float32)
    m2 = jnp.full((B, 128), INF, jnp.float32)
    m3 = jnp.full((B, 128), INF, jnp.float32)
    for c in range(N_FROM // 128):
        xb = (dbits[:, c * 128:(c + 1) * 128] & ~31) + jnp.int32(_EXP_OFF + c)
        x = lax.bitcast_convert_type(xb, jnp.float32)
        lo1 = jnp.minimum(m1, x)
        hi1 = jnp.maximum(m1, x)
        lo2 = jnp.minimum(m2, hi1)
        hi2 = jnp.maximum(m2, hi1)
        m1, m2, m3 = lo1, lo2, jnp.minimum(m3, hi2)
    cand = jnp.concatenate([m1, m2, m3], axis=1)
    pos = lax.broadcasted_iota(jnp.int32, cand.shape, 1)
    ms, lms = [], []
    for k in range(KNN):
        m = jnp.min(cand, axis=1, keepdims=True)
        lm = jnp.min(jnp.where(cand == m, pos, jnp.int32(512)),
                     axis=1, keepdims=True)
        ms.append(m)
        lms.append(lm)
        if k < KNN - 1:
            cand = jnp.where(pos == lm, INF, cand)
    mis = [lax.bitcast_convert_type(m, jnp.int32) - jnp.int32(_EXP_OFF)
           for m in ms]
    ds = [lax.bitcast_convert_type(mi & ~31, jnp.float32) for mi in mis]
    wr = [1.0 / dd for dd in ds]
    ws = wr[0] + wr[1] + wr[2]
    for k in range(KNN):
        wk = wr[k] / ws
        wk = jnp.where(jnp.isnan(wk), jnp.float32(1.0), wk)
        w_ref[:, k : k + 1] = wk
        idx_ref[:, k : k + 1] = ((mis[k] & 31) << 7) | (lms[k] & 127)


def _knn(to_aug, from_aug):
    nb = N_TO // _B_KNN
    return pl.pallas_call(
        _knn_body,
        grid=(nb,),
        in_specs=[
            pl.BlockSpec((_B_KNN, 5), lambda j: (j, 0)),
            pl.BlockSpec((5, N_FROM), lambda j: (0, 0)),
        ],
        out_specs=[
            pl.BlockSpec((_B_KNN, KNN), lambda j: (j, 0)),
            pl.BlockSpec((_B_KNN, KNN), lambda j: (j, 0)),
        ],
        out_shape=[
            jax.ShapeDtypeStruct((N_TO, KNN), jnp.int32),
            jax.ShapeDtypeStruct((N_TO, KNN), jnp.float32),
        ],
    )(to_aug, from_aug)


_NC = 2
_NS = 16
_NW = _NC * _NS
_N_ROWS = N_TO * KNN
_ROWS_PER_W = _N_ROWS // _NW
_CHUNK = 64


def _gather_sc(table, idx_flat):
    mesh = plsc.VectorSubcoreMesh(
        core_axis_name="c", subcore_axis_name="s", num_cores=_NC,
        num_subcores=_NS)

    @functools.partial(
        pl.kernel,
        out_type=jax.ShapeDtypeStruct((_N_ROWS, D_FEAT), jnp.float32),
        mesh=mesh,
        scratch_types=[
            pltpu.VMEM((_ROWS_PER_W,), jnp.int32),
            pltpu.VMEM((_CHUNK, D_FEAT), jnp.float32),
            pltpu.SemaphoreType.DMA,
            pltpu.SemaphoreType.DMA,
        ],
    )
    def k(table_hbm, idx_hbm, out_hbm, idx_v, rows_v, sem_in, sem_out):
        wid = lax.axis_index("s") * _NC + lax.axis_index("c")
        base = wid * _ROWS_PER_W
        pltpu.sync_copy(idx_hbm.at[pl.ds(base, _ROWS_PER_W)], idx_v)

        def body(i, carry):
            off = i * _CHUNK
            pltpu.async_copy(
                table_hbm.at[idx_v.at[pl.ds(off, _CHUNK)]], rows_v, sem_in
            ).wait()
            pltpu.async_copy(
                rows_v, out_hbm.at[pl.ds(base + off, _CHUNK)], sem_out
            ).wait()
            return carry

        lax.fori_loop(0, _ROWS_PER_W // _CHUNK, body, 0)

    return k(table, idx_flat)


_B_MLP = 512
_NB = N_TO // _B_MLP


def _mlp_body(tf_ref, rows_ref, w_ref,
              w0a_ref, w0b_ref, w1_ref, w2_ref,
              b0_ref, g0_ref, be0_ref, b1_ref, g1_ref, be1_ref,
              b2_ref, g2_ref, be2_ref,
              out_ref, y0_ref, y1_ref, y2_ref,
              inb_ref, inb2_ref, outb_ref, outb2_ref,
              st0_ref, st1_ref, st2_ref,
              isem, osem):
    p = pl.program_id(0)
    j = pl.program_id(1)
    s = j % 2
    last = _NB - 1

    def accum(st_ref, y):
        part = jnp.concatenate(
            [jnp.sum(y, axis=0, keepdims=True),
             jnp.sum(y * y, axis=0, keepdims=True)], axis=0)
        st_ref[...] = jnp.where(j == 0, part, st_ref[...] + part)

    def bn(y, st_ref, g_ref, be_ref):
        mean = st_ref[0:1, :] / N_TO
        var = st_ref[1:2, :] / N_TO - mean * mean
        x = g_ref[...] * (y - mean) * lax.rsqrt(var + EPS) + be_ref[...]
        return jnp.maximum(x, 0.0)

    def out_copy(buf_ref, y_ref, width, jj):
        return pltpu.make_async_copy(
            buf_ref.at[jj % 2],
            y_ref.at[pl.ds((jj % _NB) * _B_MLP, _B_MLP), pl.ds(0, width)],
            osem.at[jj % 2])

    def in_copy(y_ref, buf_ref, width, jj):
        return pltpu.make_async_copy(
            y_ref.at[pl.ds((jj % _NB) * _B_MLP, _B_MLP), pl.ds(0, width)],
            buf_ref.at[jj % 2],
            isem.at[jj % 2])

    @pl.when(p == 0)
    def _():
        w = w_ref[...]
        interp = (w[:, 0:1] * rows_ref[:, 0, :]
                  + w[:, 1:2] * rows_ref[:, 1, :]
                  + w[:, 2:3] * rows_ref[:, 2, :])
        y = (jnp.dot(tf_ref[...], w0a_ref[...],
                     preferred_element_type=jnp.float32)
             + jnp.dot(interp, w0b_ref[...],
                       preferred_element_type=jnp.float32)
             + b0_ref[...])
        accum(st0_ref, y)

        @pl.when(j >= 1)
        def _():
            out_copy(outb_ref, y0_ref, 512, j - 1).wait()
        outb_ref[s] = y
        out_copy(outb_ref, y0_ref, 512, j).start()

    @pl.when(p == 1)
    def _():
        @pl.when(j == 0)
        def _():
            out_copy(outb_ref, y0_ref, 512, last).wait()
            in_copy(y0_ref, inb_ref, 512, 0).start()
        in_copy(y0_ref, inb_ref, 512, j).wait()

        @pl.when(j < last)
        def _():
            in_copy(y0_ref, inb_ref, 512, j + 1).start()
        x = bn(inb_ref[s], st0_ref, g0_ref, be0_ref)
        y = jnp.dot(x, w1_ref[...],
                    preferred_element_type=jnp.float32) + b1_ref[...]
        accum(st1_ref, y)

        @pl.when(j >= 1)
        def _():
            out_copy(outb_ref, y1_ref, 512, j - 1).wait()
        outb_ref[s] = y
        out_copy(outb_ref, y1_ref, 512, j).start()

    @pl.when(p == 2)
    def _():
        @pl.when(j == 0)
        def _():
            out_copy(outb_ref, y1_ref, 512, last).wait()
            in_copy(y1_ref, inb_ref, 512, 0).start()
        in_copy(y1_ref, inb_ref, 512, j).wait()

        @pl.when(j < last)
        def _():
            in_copy(y1_ref, inb_ref, 512, j + 1).start()
        x = bn(inb_ref[s], st1_ref, g1_ref, be1_ref)
        y = jnp.dot(x, w2_ref[...],
                    preferred_element_type=jnp.float32) + b2_ref[...]
        accum(st2_ref, y)

        @pl.when(j >= 1)
        def _():
            out_copy(outb2_ref, y2_ref, 256, j - 1).wait()
        outb2_ref[s] = y
        out_copy(outb2_ref, y2_ref, 256, j).start()

    @pl.when(p == 3)
    def _():
        @pl.when(j == 0)
        def _():
            out_copy(outb2_ref, y2_ref, 256, last).wait()
            in_copy(y2_ref, inb2_ref, 256, 0).start()
        in_copy(y2_ref, inb2_ref, 256, j).wait()

        @pl.when(j < last)
        def _():
            in_copy(y2_ref, inb2_ref, 256, j + 1).start()
        out_ref[...] = bn(inb2_ref[s], st2_ref, g2_ref, be2_ref)


def _mlp(to_features, rows, w, w0aT, w0bT, w1T, w2T,
         b0, g0, be0, b1, g1, be1, b2, g2, be2):
    def blk(jx):
        return lambda p, j: (jnp.where(p == jx, j, 0), 0)

    def blk3(jx):
        return lambda p, j: (jnp.where(p == jx, j, 0), 0, 0)

    def const2(p, j):
        return (0, 0)

    outs = pl.pallas_call(
        _mlp_body,
        grid=(4, _NB),
        in_specs=[
            pl.BlockSpec((_B_MLP, D_TO), blk(0)),
            pl.BlockSpec((_B_MLP, KNN, D_FEAT), blk3(0)),
            pl.BlockSpec((_B_MLP, KNN), blk(0)),
            pl.BlockSpec((D_TO, 512), const2),
            pl.BlockSpec((D_FEAT, 512), const2),
            pl.BlockSpec((512, 512), const2),
            pl.BlockSpec((512, 256), const2),
        ] + [pl.BlockSpec((1, 512), const2)] * 6
          + [pl.BlockSpec((1, 256), const2)] * 3,
        out_specs=[
            pl.BlockSpec((_B_MLP, 256), blk(3)),
            pl.BlockSpec(memory_space=pltpu.MemorySpace.HBM),
            pl.BlockSpec(memory_space=pltpu.MemorySpace.HBM),
            pl.BlockSpec(memory_space=pltpu.MemorySpace.HBM),
        ],
        out_shape=[
            jax.ShapeDtypeStruct((N_TO, 256), jnp.float32),
            jax.ShapeDtypeStruct((N_TO, 512), jnp.float32),
            jax.ShapeDtypeStruct((N_TO, 512), jnp.float32),
            jax.ShapeDtypeStruct((N_TO, 256), jnp.float32),
        ],
        scratch_shapes=[
            pltpu.VMEM((2, _B_MLP, 512), jnp.float32),
            pltpu.VMEM((2, _B_MLP, 256), jnp.float32),
            pltpu.VMEM((2, _B_MLP, 512), jnp.float32),
            pltpu.VMEM((2, _B_MLP, 256), jnp.float32),
            pltpu.VMEM((2, 512), jnp.float32),
            pltpu.VMEM((2, 512), jnp.float32),
            pltpu.VMEM((2, 256), jnp.float32),
            pltpu.SemaphoreType.DMA((2,)),
            pltpu.SemaphoreType.DMA((2,)),
        ],
    )(to_features, rows, w, w0aT, w0bT, w1T, w2T,
      b0, g0, be0, b1, g1, be1, b2, g2, be2)
    return outs[0]


def kernel(from_coords, from_features, to_coords, to_features,
           W0, b0, g0, be0, W1, b1, g1, be1, W2, b2, g2, be2):
    to_aug = jnp.concatenate(
        [-2.0 * to_coords,
         jnp.sum(to_coords * to_coords, axis=1, keepdims=True),
         jnp.ones((N_TO, 1), jnp.float32)], axis=1)
    from_aug = jnp.concatenate(
        [from_coords.T,
         jnp.ones((1, N_FROM), jnp.float32),
         jnp.sum(from_coords * from_coords, axis=1)[None, :]], axis=0)
    idx, w = _knn(to_aug, from_aug)
    rows = _gather_sc(from_features, idx.reshape(-1))
    rows = rows.reshape(N_TO, KNN, D_FEAT)
    return _mlp(to_features, rows, w,
                W0[:, :D_TO].T, W0[:, D_TO:].T, W1.T, W2.T,
                b0[None, :], g0[None, :], be0[None, :],
                b1[None, :], g1[None, :], be1[None, :],
                b2[None, :], g2[None, :], be2[None, :])


# speedup vs baseline: 1.4433x; 1.4433x over previous
import functools

import jax
import jax.numpy as jnp
from jax import lax
from jax.experimental import pallas as pl
from jax.experimental.pallas import tpu as pltpu
from jax.experimental.pallas import tpu_sc as plsc

N_TO = 16384
N_FROM = 4096
KNN = 3
D_FEAT = 512
D_TO = 256
EPS = 1e-5


_B_KNN = 512
_EXP_OFF = 64 << 23


def _knn_body(to_ref, fromT_ref, idx_ref, w_ref):
    t = to_ref[...]
    f = fromT_ref[...]
    diff = t[:, 0:1] - f[0:1, :]
    d = diff * diff
    diff = t[:, 1:2] - f[1:2, :]
    d = d + diff * diff
    diff = t[:, 2:3] - f[2:3, :]
    d = d + diff * diff
    dbits = lax.bitcast_convert_type(d, jnp.int32)
    INF = jnp.float32(jnp.inf)
    B = d.shape[0]
    m1 = jnp.full((B, 128), INF, jnp.float32)
    m2 = jnp.full((B, 128), INF, jnp.float32)
    m3 = jnp.full((B, 128), INF, jnp.float32)
    for c in range(N_FROM // 128):
        xb = (dbits[:, c * 128:(c + 1) * 128] & ~31) + jnp.int32(_EXP_OFF + c)
        x = lax.bitcast_convert_type(xb, jnp.float32)
        lo1 = jnp.minimum(m1, x)
        hi1 = jnp.maximum(m1, x)
        lo2 = jnp.minimum(m2, hi1)
        hi2 = jnp.maximum(m2, hi1)
        m1, m2, m3 = lo1, lo2, jnp.minimum(m3, hi2)
    cand = jnp.concatenate([m1, m2, m3], axis=1)
    pos = lax.broadcasted_iota(jnp.int32, cand.shape, 1)
    ms, lms = [], []
    for k in range(KNN):
        m = jnp.min(cand, axis=1, keepdims=True)
        lm = jnp.min(jnp.where(cand == m, pos, jnp.int32(512)),
                     axis=1, keepdims=True)
        ms.append(m)
        lms.append(lm)
        if k < KNN - 1:
            cand = jnp.where(pos == lm, INF, cand)
    mis = [lax.bitcast_convert_type(m, jnp.int32) - jnp.int32(_EXP_OFF)
           for m in ms]
    ds = [lax.bitcast_convert_type(mi & ~31, jnp.float32) for mi in mis]
    wr = [1.0 / dd for dd in ds]
    ws = wr[0] + wr[1] + wr[2]
    for k in range(KNN):
        wk = wr[k] / ws
        wk = jnp.where(jnp.isnan(wk), jnp.float32(1.0), wk)
        w_ref[:, k : k + 1] = wk
        idx_ref[:, k : k + 1] = ((mis[k] & 31) << 7) | (lms[k] & 127)


def _knn(to_coords, fromT):
    nb = N_TO // _B_KNN
    return pl.pallas_call(
        _knn_body,
        grid=(nb,),
        in_specs=[
            pl.BlockSpec((_B_KNN, 3), lambda j: (j, 0)),
            pl.BlockSpec((3, N_FROM), lambda j: (0, 0)),
        ],
        out_specs=[
            pl.BlockSpec((_B_KNN, KNN), lambda j: (j, 0)),
            pl.BlockSpec((_B_KNN, KNN), lambda j: (j, 0)),
        ],
        out_shape=[
            jax.ShapeDtypeStruct((N_TO, KNN), jnp.int32),
            jax.ShapeDtypeStruct((N_TO, KNN), jnp.float32),
        ],
    )(to_coords, fromT)


_NC = 2
_NS = 16
_NW = _NC * _NS
_N_ROWS = N_TO * KNN
_ROWS_PER_W = _N_ROWS // _NW
_CHUNK = 64


def _gather_sc(table, idx_flat):
    mesh = plsc.VectorSubcoreMesh(
        core_axis_name="c", subcore_axis_name="s", num_cores=_NC,
        num_subcores=_NS)

    @functools.partial(
        pl.kernel,
        out_type=jax.ShapeDtypeStruct((_N_ROWS, D_FEAT), jnp.float32),
        mesh=mesh,
        scratch_types=[
            pltpu.VMEM((_ROWS_PER_W,), jnp.int32),
            pltpu.VMEM((_CHUNK, D_FEAT), jnp.float32),
            pltpu.SemaphoreType.DMA,
            pltpu.SemaphoreType.DMA,
        ],
    )
    def k(table_hbm, idx_hbm, out_hbm, idx_v, rows_v, sem_in, sem_out):
        wid = lax.axis_index("s") * _NC + lax.axis_index("c")
        base = wid * _ROWS_PER_W
        pltpu.sync_copy(idx_hbm.at[pl.ds(base, _ROWS_PER_W)], idx_v)

        def body(i, carry):
            off = i * _CHUNK
            pltpu.async_copy(
                table_hbm.at[idx_v.at[pl.ds(off, _CHUNK)]], rows_v, sem_in
            ).wait()
            pltpu.async_copy(
                rows_v, out_hbm.at[pl.ds(base + off, _CHUNK)], sem_out
            ).wait()
            return carry

        lax.fori_loop(0, _ROWS_PER_W // _CHUNK, body, 0)

    return k(table, idx_flat)


_B_MLP = 512
_NB = N_TO // _B_MLP


def _mlp_body(tf_ref, rows_ref, w_ref,
              w0a_ref, w0b_ref, w1_ref, w2_ref,
              b0_ref, g0_ref, be0_ref, b1_ref, g1_ref, be1_ref,
              b2_ref, g2_ref, be2_ref,
              out_ref, y0_ref, y1_ref, y2_ref,
              inb_ref, inb2_ref, outb_ref, outb2_ref,
              st0_ref, st1_ref, st2_ref,
              isem, osem):
    p = pl.program_id(0)
    j = pl.program_id(1)
    s = j % 2
    last = _NB - 1

    def accum(st_ref, y):
        part = jnp.concatenate(
            [jnp.sum(y, axis=0, keepdims=True),
             jnp.sum(y * y, axis=0, keepdims=True)], axis=0)
        st_ref[...] = jnp.where(j == 0, part, st_ref[...] + part)

    def bn(y, st_ref, g_ref, be_ref):
        mean = st_ref[0:1, :] / N_TO
        var = st_ref[1:2, :] / N_TO - mean * mean
        x = g_ref[...] * (y - mean) * lax.rsqrt(var + EPS) + be_ref[...]
        return jnp.maximum(x, 0.0)

    def out_copy(buf_ref, y_ref, width, jj):
        return pltpu.make_async_copy(
            buf_ref.at[jj % 2],
            y_ref.at[pl.ds((jj % _NB) * _B_MLP, _B_MLP), pl.ds(0, width)],
            osem.at[jj % 2])

    def in_copy(y_ref, buf_ref, width, jj):
        return pltpu.make_async_copy(
            y_ref.at[pl.ds((jj % _NB) * _B_MLP, _B_MLP), pl.ds(0, width)],
            buf_ref.at[jj % 2],
            isem.at[jj % 2])

    @pl.when(p == 0)
    def _():
        w = w_ref[...]
        interp = (w[:, 0:1] * rows_ref[:, 0, :]
                  + w[:, 1:2] * rows_ref[:, 1, :]
                  + w[:, 2:3] * rows_ref[:, 2, :])
        y = (jnp.dot(tf_ref[...], w0a_ref[...],
                     preferred_element_type=jnp.float32)
             + jnp.dot(interp, w0b_ref[...],
                       preferred_element_type=jnp.float32)
             + b0_ref[...])
        accum(st0_ref, y)

        @pl.when(j >= 1)
        def _():
            out_copy(outb_ref, y0_ref, 512, j - 1).wait()
        outb_ref[s] = y
        out_copy(outb_ref, y0_ref, 512, j).start()

    @pl.when(p == 1)
    def _():
        @pl.when(j == 0)
        def _():
            out_copy(outb_ref, y0_ref, 512, last).wait()
            in_copy(y0_ref, inb_ref, 512, 0).start()
        in_copy(y0_ref, inb_ref, 512, j).wait()

        @pl.when(j < last)
        def _():
            in_copy(y0_ref, inb_ref, 512, j + 1).start()
        x = bn(inb_ref[s], st0_ref, g0_ref, be0_ref)
        y = jnp.dot(x, w1_ref[...],
                    preferred_element_type=jnp.float32) + b1_ref[...]
        accum(st1_ref, y)

        @pl.when(j >= 1)
        def _():
            out_copy(outb_ref, y1_ref, 512, j - 1).wait()
        outb_ref[s] = y
        out_copy(outb_ref, y1_ref, 512, j).start()

    @pl.when(p == 2)
    def _():
        @pl.when(j == 0)
        def _():
            out_copy(outb_ref, y1_ref, 512, last).wait()
            in_copy(y1_ref, inb_ref, 512, 0).start()
        in_copy(y1_ref, inb_ref, 512, j).wait()

        @pl.when(j < last)
        def _():
            in_copy(y1_ref, inb_ref, 512, j + 1).start()
        x = bn(inb_ref[s], st1_ref, g1_ref, be1_ref)
        y = jnp.dot(x, w2_ref[...],
                    preferred_element_type=jnp.float32) + b2_ref[...]
        accum(st2_ref, y)

        @pl.when(j >= 1)
        def _():
            out_copy(outb2_ref, y2_ref, 256, j - 1).wait()
        outb2_ref[s] = y
        out_copy(outb2_ref, y2_ref, 256, j).start()

    @pl.when(p == 3)
    def _():
        @pl.when(j == 0)
        def _():
            out_copy(outb2_ref, y2_ref, 256, last).wait()
            in_copy(y2_ref, inb2_ref, 256, 0).start()
        in_copy(y2_ref, inb2_ref, 256, j).wait()

        @pl.when(j < last)
        def _():
            in_copy(y2_ref, inb2_ref, 256, j + 1).start()
        out_ref[...] = bn(inb2_ref[s], st2_ref, g2_ref, be2_ref)


def _mlp(to_features, rows, w, w0aT, w0bT, w1T, w2T,
         b0, g0, be0, b1, g1, be1, b2, g2, be2):
    def blk(jx):
        return lambda p, j: (jnp.where(p == jx, j, 0), 0)

    def blk3(jx):
        return lambda p, j: (jnp.where(p == jx, j, 0), 0, 0)

    def const2(p, j):
        return (0, 0)

    outs = pl.pallas_call(
        _mlp_body,
        grid=(4, _NB),
        in_specs=[
            pl.BlockSpec((_B_MLP, D_TO), blk(0)),
            pl.BlockSpec((_B_MLP, KNN, D_FEAT), blk3(0)),
            pl.BlockSpec((_B_MLP, KNN), blk(0)),
            pl.BlockSpec((D_TO, 512), const2),
            pl.BlockSpec((D_FEAT, 512), const2),
            pl.BlockSpec((512, 512), const2),
            pl.BlockSpec((512, 256), const2),
        ] + [pl.BlockSpec((1, 512), const2)] * 6
          + [pl.BlockSpec((1, 256), const2)] * 3,
        out_specs=[
            pl.BlockSpec((_B_MLP, 256), blk(3)),
            pl.BlockSpec(memory_space=pltpu.MemorySpace.HBM),
            pl.BlockSpec(memory_space=pltpu.MemorySpace.HBM),
            pl.BlockSpec(memory_space=pltpu.MemorySpace.HBM),
        ],
        out_shape=[
            jax.ShapeDtypeStruct((N_TO, 256), jnp.float32),
            jax.ShapeDtypeStruct((N_TO, 512), jnp.float32),
            jax.ShapeDtypeStruct((N_TO, 512), jnp.float32),
            jax.ShapeDtypeStruct((N_TO, 256), jnp.float32),
        ],
        scratch_shapes=[
            pltpu.VMEM((2, _B_MLP, 512), jnp.float32),
            pltpu.VMEM((2, _B_MLP, 256), jnp.float32),
            pltpu.VMEM((2, _B_MLP, 512), jnp.float32),
            pltpu.VMEM((2, _B_MLP, 256), jnp.float32),
            pltpu.VMEM((2, 512), jnp.float32),
            pltpu.VMEM((2, 512), jnp.float32),
            pltpu.VMEM((2, 256), jnp.float32),
            pltpu.SemaphoreType.DMA((2,)),
            pltpu.SemaphoreType.DMA((2,)),
        ],
    )(to_features, rows, w, w0aT, w0bT, w1T, w2T,
      b0, g0, be0, b1, g1, be1, b2, g2, be2)
    return outs[0]


def kernel(from_coords, from_features, to_coords, to_features,
           W0, b0, g0, be0, W1, b1, g1, be1, W2, b2, g2, be2):
    idx, w = _knn(to_coords, from_coords.T)
    rows = _gather_sc(from_features, idx.reshape(-1))
    rows = rows.reshape(N_TO, KNN, D_FEAT)
    return _mlp(to_features, rows, w,
                W0[:, :D_TO].T, W0[:, D_TO:].T, W1.T, W2.T,
                b0[None, :], g0[None, :], be0[None, :],
                b1[None, :], g1[None, :], be1[None, :],
                b2[None, :], g2[None, :], be2[None, :])


# speedup vs baseline: 1.4440x; 1.0005x over previous
import functools

import jax
import jax.numpy as jnp
from jax import lax
from jax.experimental import pallas as pl
from jax.experimental.pallas import tpu as pltpu
from jax.experimental.pallas import tpu_sc as plsc

N_TO = 16384
N_FROM = 4096
KNN = 3
D_FEAT = 512
D_TO = 256
EPS = 1e-5


_B_KNN = 512
_EXP_OFF = 64 << 23


def _knn_body(to_ref, fromT_ref, idx_ref, w_ref):
    t = to_ref[...]
    f = fromT_ref[...]
    diff = t[:, 0:1] - f[0:1, :]
    d = diff * diff
    diff = t[:, 1:2] - f[1:2, :]
    d = d + diff * diff
    diff = t[:, 2:3] - f[2:3, :]
    d = d + diff * diff
    dbits = lax.bitcast_convert_type(d, jnp.int32)
    INF = jnp.float32(jnp.inf)
    B = d.shape[0]
    m1 = jnp.full((B, 128), INF, jnp.float32)
    m2 = jnp.full((B, 128), INF, jnp.float32)
    m3 = jnp.full((B, 128), INF, jnp.float32)
    for c in range(N_FROM // 128):
        xb = (dbits[:, c * 128:(c + 1) * 128] & ~31) + jnp.int32(_EXP_OFF + c)
        x = lax.bitcast_convert_type(xb, jnp.float32)
        lo1 = jnp.minimum(m1, x)
        hi1 = jnp.maximum(m1, x)
        lo2 = jnp.minimum(m2, hi1)
        hi2 = jnp.maximum(m2, hi1)
        m1, m2, m3 = lo1, lo2, jnp.minimum(m3, hi2)
    cand = jnp.concatenate([m1, m2, m3], axis=1)
    pos = lax.broadcasted_iota(jnp.int32, cand.shape, 1)
    ms, lms = [], []
    for k in range(KNN):
        m = jnp.min(cand, axis=1, keepdims=True)
        lm = jnp.min(jnp.where(cand == m, pos, jnp.int32(512)),
                     axis=1, keepdims=True)
        ms.append(m)
        lms.append(lm)
        if k < KNN - 1:
            cand = jnp.where(pos == lm, INF, cand)
    mis = [lax.bitcast_convert_type(m, jnp.int32) - jnp.int32(_EXP_OFF)
           for m in ms]
    ds = [lax.bitcast_convert_type(mi & ~31, jnp.float32) for mi in mis]
    wr = [1.0 / dd for dd in ds]
    ws = wr[0] + wr[1] + wr[2]
    for k in range(KNN):
        wk = wr[k] / ws
        wk = jnp.where(jnp.isnan(wk), jnp.float32(1.0), wk)
        w_ref[:, k : k + 1] = wk
        idx_ref[:, k : k + 1] = ((mis[k] & 31) << 7) | (lms[k] & 127)


def _knn(to_coords, fromT):
    nb = N_TO // _B_KNN
    return pl.pallas_call(
        _knn_body,
        grid=(nb,),
        in_specs=[
            pl.BlockSpec((_B_KNN, 3), lambda j: (j, 0)),
            pl.BlockSpec((3, N_FROM), lambda j: (0, 0)),
        ],
        out_specs=[
            pl.BlockSpec((_B_KNN, KNN), lambda j: (j, 0)),
            pl.BlockSpec((_B_KNN, KNN), lambda j: (j, 0)),
        ],
        out_shape=[
            jax.ShapeDtypeStruct((N_TO, KNN), jnp.int32),
            jax.ShapeDtypeStruct((N_TO, KNN), jnp.float32),
        ],
    )(to_coords, fromT)


_NC = 2
_NS = 16
_NW = _NC * _NS
_N_ROWS = N_TO * KNN
_ROWS_PER_W = _N_ROWS // _NW
_CHUNK = 64


_PTS_PER_W = N_TO // _NW
_PCH = 16
_NCH = _PTS_PER_W // _PCH


def _interp_sc(table, idx_flat, w_flat):
    mesh = plsc.VectorSubcoreMesh(
        core_axis_name="c", subcore_axis_name="s", num_cores=_NC,
        num_subcores=_NS)

    @functools.partial(
        pl.kernel,
        out_type=jax.ShapeDtypeStruct((N_TO, D_FEAT), jnp.float32),
        mesh=mesh,
        compiler_params=pltpu.CompilerParams(needs_layout_passes=False),
        scratch_types=[
            pltpu.VMEM((_ROWS_PER_W,), jnp.int32),
            pltpu.VMEM((_ROWS_PER_W,), jnp.float32),
            pltpu.VMEM((2, 3 * _PCH, D_FEAT), jnp.float32),
            pltpu.VMEM((2, _PCH, D_FEAT), jnp.float32),
            pltpu.SemaphoreType.DMA,
            pltpu.SemaphoreType.DMA,
            pltpu.SemaphoreType.DMA,
            pltpu.SemaphoreType.DMA,
        ],
    )
    def k(table_hbm, idx_hbm, w_hbm, out_hbm, idx_v, w_v, rows_v, out_v,
          gsem0, gsem1, osem0, osem1):
        wid = lax.axis_index("s") * _NC + lax.axis_index("c")
        rbase = wid * _ROWS_PER_W
        pbase = wid * _PTS_PER_W
        pltpu.sync_copy(idx_hbm.at[pl.ds(rbase, _ROWS_PER_W)], idx_v)
        pltpu.sync_copy(w_hbm.at[pl.ds(rbase, _ROWS_PER_W)], w_v)

        def gather_start(g, slot, sem):
            pltpu.async_copy(
                table_hbm.at[idx_v.at[pl.ds(g * 3 * _PCH, 3 * _PCH)]],
                rows_v.at[slot], sem)

        def gather_wait(slot, sem):
            pltpu.make_async_copy(
                table_hbm.at[pl.ds(0, 3 * _PCH)], rows_v.at[slot], sem
            ).wait()

        def out_desc(g, slot, sem):
            return pltpu.make_async_copy(
                out_v.at[slot],
                out_hbm.at[pl.ds(pbase + g * _PCH, _PCH)], sem)

        def compute(g, slot):
            def pbody(pp, carry):
                base = g * (3 * _PCH) + 3 * pp
                iv = jnp.zeros((16,), jnp.int32) + base
                w0 = plsc.load_gather(w_v, [iv])
                w1 = plsc.load_gather(w_v, [iv + 1])
                w2 = plsc.load_gather(w_v, [iv + 2])
                for cc in range(D_FEAT // 16):
                    sl = pl.ds(cc * 16, 16)
                    acc = (rows_v[slot, 3 * pp, sl] * w0
                           + rows_v[slot, 3 * pp + 1, sl] * w1
                           + rows_v[slot, 3 * pp + 2, sl] * w2)
                    out_v[slot, pp, sl] = acc
                return carry
            lax.fori_loop(0, _PCH, pbody, 0)

        gather_start(0, 0, gsem0)
        gather_start(1, 1, gsem1)

        def body(h, carry):
            g0 = 2 * h
            g1 = 2 * h + 1

            gather_wait(0, gsem0)

            @pl.when(h > 0)
            def _():
                out_desc(g0 - 2, 0, osem0).wait()
            compute(g0, 0)
            out_desc(g0, 0, osem0).start()

            @pl.when(g0 + 2 < _NCH)
            def _():
                gather_start(g0 + 2, 0, gsem0)

            gather_wait(1, gsem1)

            @pl.when(h > 0)
            def _():
                out_desc(g1 - 2, 1, osem1).wait()
            compute(g1, 1)
            out_desc(g1, 1, osem1).start()

            @pl.when(g1 + 2 < _NCH)
            def _():
                gather_start(g1 + 2, 1, gsem1)
            return carry

        lax.fori_loop(0, _NCH // 2, body, 0)
        out_desc(_NCH - 2, 0, osem0).wait()
        out_desc(_NCH - 1, 1, osem1).wait()

    return k(table, idx_flat, w_flat)


_B_MLP = 512
_NB = N_TO // _B_MLP


def _mlp_body(tf_ref, interp_ref,
              w0a_ref, w0b_ref, w1_ref, w2_ref,
              b0_ref, g0_ref, be0_ref, b1_ref, g1_ref, be1_ref,
              b2_ref, g2_ref, be2_ref,
              out_ref, y0_ref, y1_ref, y2_ref,
              inb_ref, inb2_ref, outb_ref, outb2_ref,
              st0_ref, st1_ref, st2_ref,
              isem, osem):
    p = pl.program_id(0)
    j = pl.program_id(1)
    s = j % 2
    last = _NB - 1

    def accum(st_ref, y):
        part = jnp.concatenate(
            [jnp.sum(y, axis=0, keepdims=True),
             jnp.sum(y * y, axis=0, keepdims=True)], axis=0)
        st_ref[...] = jnp.where(j == 0, part, st_ref[...] + part)

    def bn(y, st_ref, g_ref, be_ref):
        mean = st_ref[0:1, :] / N_TO
        var = st_ref[1:2, :] / N_TO - mean * mean
        x = g_ref[...] * (y - mean) * lax.rsqrt(var + EPS) + be_ref[...]
        return jnp.maximum(x, 0.0)

    def out_copy(buf_ref, y_ref, width, jj):
        return pltpu.make_async_copy(
            buf_ref.at[jj % 2],
            y_ref.at[pl.ds((jj % _NB) * _B_MLP, _B_MLP), pl.ds(0, width)],
            osem.at[jj % 2])

    def in_copy(y_ref, buf_ref, width, jj):
        return pltpu.make_async_copy(
            y_ref.at[pl.ds((jj % _NB) * _B_MLP, _B_MLP), pl.ds(0, width)],
            buf_ref.at[jj % 2],
            isem.at[jj % 2])

    @pl.when(p == 0)
    def _():
        y = (jnp.dot(tf_ref[...], w0a_ref[...],
                     preferred_element_type=jnp.float32)
             + jnp.dot(interp_ref[...], w0b_ref[...],
                       preferred_element_type=jnp.float32)
             + b0_ref[...])
        accum(st0_ref, y)

        @pl.when(j >= 1)
        def _():
            out_copy(outb_ref, y0_ref, 512, j - 1).wait()
        outb_ref[s] = y
        out_copy(outb_ref, y0_ref, 512, j).start()

    @pl.when(p == 1)
    def _():
        @pl.when(j == 0)
        def _():
            out_copy(outb_ref, y0_ref, 512, last).wait()
            in_copy(y0_ref, inb_ref, 512, 0).start()
        in_copy(y0_ref, inb_ref, 512, j).wait()

        @pl.when(j < last)
        def _():
            in_copy(y0_ref, inb_ref, 512, j + 1).start()
        x = bn(inb_ref[s], st0_ref, g0_ref, be0_ref)
        y = jnp.dot(x, w1_ref[...],
                    preferred_element_type=jnp.float32) + b1_ref[...]
        accum(st1_ref, y)

        @pl.when(j >= 1)
        def _():
            out_copy(outb_ref, y1_ref, 512, j - 1).wait()
        outb_ref[s] = y
        out_copy(outb_ref, y1_ref, 512, j).start()

    @pl.when(p == 2)
    def _():
        @pl.when(j == 0)
        def _():
            out_copy(outb_ref, y1_ref, 512, last).wait()
            in_copy(y1_ref, inb_ref, 512, 0).start()
        in_copy(y1_ref, inb_ref, 512, j).wait()

        @pl.when(j < last)
        def _():
            in_copy(y1_ref, inb_ref, 512, j + 1).start()
        x = bn(inb_ref[s], st1_ref, g1_ref, be1_ref)
        y = jnp.dot(x, w2_ref[...],
                    preferred_element_type=jnp.float32) + b2_ref[...]
        accum(st2_ref, y)

        @pl.when(j >= 1)
        def _():
            out_copy(outb2_ref, y2_ref, 256, j - 1).wait()
        outb2_ref[s] = y
        out_copy(outb2_ref, y2_ref, 256, j).start()

    @pl.when(p == 3)
    def _():
        @pl.when(j == 0)
        def _():
            out_copy(outb2_ref, y2_ref, 256, last).wait()
            in_copy(y2_ref, inb2_ref, 256, 0).start()
        in_copy(y2_ref, inb2_ref, 256, j).wait()

        @pl.when(j < last)
        def _():
            in_copy(y2_ref, inb2_ref, 256, j + 1).start()
        out_ref[...] = bn(inb2_ref[s], st2_ref, g2_ref, be2_ref)


def _mlp(to_features, interp, w0aT, w0bT, w1T, w2T,
         b0, g0, be0, b1, g1, be1, b2, g2, be2):
    def blk(jx):
        return lambda p, j: (jnp.where(p == jx, j, 0), 0)

    def const2(p, j):
        return (0, 0)

    outs = pl.pallas_call(
        _mlp_body,
        grid=(4, _NB),
        in_specs=[
            pl.BlockSpec((_B_MLP, D_TO), blk(0)),
            pl.BlockSpec((_B_MLP, D_FEAT), blk(0)),
            pl.BlockSpec((D_TO, 512), const2),
            pl.BlockSpec((D_FEAT, 512), const2),
            pl.BlockSpec((512, 512), const2),
            pl.BlockSpec((512, 256), const2),
        ] + [pl.BlockSpec((1, 512), const2)] * 6
          + [pl.BlockSpec((1, 256), const2)] * 3,
        out_specs=[
            pl.BlockSpec((_B_MLP, 256), blk(3)),
            pl.BlockSpec(memory_space=pltpu.MemorySpace.HBM),
            pl.BlockSpec(memory_space=pltpu.MemorySpace.HBM),
            pl.BlockSpec(memory_space=pltpu.MemorySpace.HBM),
        ],
        out_shape=[
            jax.ShapeDtypeStruct((N_TO, 256), jnp.float32),
            jax.ShapeDtypeStruct((N_TO, 512), jnp.float32),
            jax.ShapeDtypeStruct((N_TO, 512), jnp.float32),
            jax.ShapeDtypeStruct((N_TO, 256), jnp.float32),
        ],
        scratch_shapes=[
            pltpu.VMEM((2, _B_MLP, 512), jnp.float32),
            pltpu.VMEM((2, _B_MLP, 256), jnp.float32),
            pltpu.VMEM((2, _B_MLP, 512), jnp.float32),
            pltpu.VMEM((2, _B_MLP, 256), jnp.float32),
            pltpu.VMEM((2, 512), jnp.float32),
            pltpu.VMEM((2, 512), jnp.float32),
            pltpu.VMEM((2, 256), jnp.float32),
            pltpu.SemaphoreType.DMA((2,)),
            pltpu.SemaphoreType.DMA((2,)),
        ],
    )(to_features, interp, w0aT, w0bT, w1T, w2T,
      b0, g0, be0, b1, g1, be1, b2, g2, be2)
    return outs[0]


def kernel(from_coords, from_features, to_coords, to_features,
           W0, b0, g0, be0, W1, b1, g1, be1, W2, b2, g2, be2):
    idx, w = _knn(to_coords, from_coords.T)
    interp = _interp_sc(from_features, idx.reshape(-1), w.reshape(-1))
    return _mlp(to_features, interp,
                W0[:, :D_TO].T, W0[:, D_TO:].T, W1.T, W2.T,
                b0[None, :], g0[None, :], be0[None, :],
                b1[None, :], g1[None, :], be1[None, :],
                b2[None, :], g2[None, :], be2[None, :])


# speedup vs baseline: 1.6145x; 1.1181x over previous
import functools

import jax
import jax.numpy as jnp
from jax import lax
from jax.experimental import pallas as pl
from jax.experimental.pallas import tpu as pltpu
from jax.experimental.pallas import tpu_sc as plsc

N_TO = 16384
N_FROM = 4096
KNN = 3
D_FEAT = 512
D_TO = 256
EPS = 1e-5


_B_KNN = 512
_EXP_OFF = 64 << 23


def _knn_body(to_ref, fromT_ref, idx_ref, w_ref):
    t = to_ref[...]
    f = fromT_ref[...]
    diff = t[:, 0:1] - f[0:1, :]
    d = diff * diff
    diff = t[:, 1:2] - f[1:2, :]
    d = d + diff * diff
    diff = t[:, 2:3] - f[2:3, :]
    d = d + diff * diff
    dbits = lax.bitcast_convert_type(d, jnp.int32)
    INF = jnp.float32(jnp.inf)
    B = d.shape[0]
    m1 = jnp.full((B, 128), INF, jnp.float32)
    m2 = jnp.full((B, 128), INF, jnp.float32)
    m3 = jnp.full((B, 128), INF, jnp.float32)
    for c in range(N_FROM // 128):
        xb = (dbits[:, c * 128:(c + 1) * 128] & ~31) + jnp.int32(_EXP_OFF + c)
        x = lax.bitcast_convert_type(xb, jnp.float32)
        lo1 = jnp.minimum(m1, x)
        hi1 = jnp.maximum(m1, x)
        lo2 = jnp.minimum(m2, hi1)
        hi2 = jnp.maximum(m2, hi1)
        m1, m2, m3 = lo1, lo2, jnp.minimum(m3, hi2)
    cand = jnp.concatenate([m1, m2, m3], axis=1)
    pos = lax.broadcasted_iota(jnp.int32, cand.shape, 1)
    ms, lms = [], []
    for k in range(KNN):
        m = jnp.min(cand, axis=1, keepdims=True)
        lm = jnp.min(jnp.where(cand == m, pos, jnp.int32(512)),
                     axis=1, keepdims=True)
        ms.append(m)
        lms.append(lm)
        if k < KNN - 1:
            cand = jnp.where(pos == lm, INF, cand)
    mis = [lax.bitcast_convert_type(m, jnp.int32) - jnp.int32(_EXP_OFF)
           for m in ms]
    ds = [lax.bitcast_convert_type(mi & ~31, jnp.float32) for mi in mis]
    wr = [1.0 / dd for dd in ds]
    ws = wr[0] + wr[1] + wr[2]
    for k in range(KNN):
        wk = wr[k] / ws
        wk = jnp.where(jnp.isnan(wk), jnp.float32(1.0), wk)
        w_ref[:, k : k + 1] = wk
        idx_ref[:, k : k + 1] = ((mis[k] & 31) << 7) | (lms[k] & 127)


def _knn(to_coords, fromT):
    n = to_coords.shape[0]
    nb = n // _B_KNN
    return pl.pallas_call(
        _knn_body,
        grid=(nb,),
        in_specs=[
            pl.BlockSpec((_B_KNN, 3), lambda j: (j, 0)),
            pl.BlockSpec((3, N_FROM), lambda j: (0, 0)),
        ],
        out_specs=[
            pl.BlockSpec((_B_KNN, KNN), lambda j: (j, 0)),
            pl.BlockSpec((_B_KNN, KNN), lambda j: (j, 0)),
        ],
        out_shape=[
            jax.ShapeDtypeStruct((n, KNN), jnp.int32),
            jax.ShapeDtypeStruct((n, KNN), jnp.float32),
        ],
    )(to_coords, fromT)


_NC = 2
_NS = 16
_NW = _NC * _NS
_N_ROWS = N_TO * KNN
_ROWS_PER_W = _N_ROWS // _NW
_CHUNK = 64


_PCH = 16


def _interp_sc(table, idx_flat, w_flat):
    n_pts = idx_flat.shape[0] // KNN
    pts_per_w = n_pts // _NW
    rows_per_w = KNN * pts_per_w
    nch = pts_per_w // _PCH
    mesh = plsc.VectorSubcoreMesh(
        core_axis_name="c", subcore_axis_name="s", num_cores=_NC,
        num_subcores=_NS)

    @functools.partial(
        pl.kernel,
        out_type=jax.ShapeDtypeStruct((n_pts, D_FEAT), jnp.float32),
        mesh=mesh,
        compiler_params=pltpu.CompilerParams(needs_layout_passes=False),
        scratch_types=[
            pltpu.VMEM((rows_per_w,), jnp.int32),
            pltpu.VMEM((rows_per_w,), jnp.float32),
            pltpu.VMEM((2, 3 * _PCH, D_FEAT), jnp.float32),
            pltpu.VMEM((2, _PCH, D_FEAT), jnp.float32),
            pltpu.SemaphoreType.DMA,
            pltpu.SemaphoreType.DMA,
            pltpu.SemaphoreType.DMA,
            pltpu.SemaphoreType.DMA,
        ],
    )
    def k(table_hbm, idx_hbm, w_hbm, out_hbm, idx_v, w_v, rows_v, out_v,
          gsem0, gsem1, osem0, osem1):
        wid = lax.axis_index("s") * _NC + lax.axis_index("c")
        rbase = wid * rows_per_w
        pbase = wid * pts_per_w
        pltpu.sync_copy(idx_hbm.at[pl.ds(rbase, rows_per_w)], idx_v)
        pltpu.sync_copy(w_hbm.at[pl.ds(rbase, rows_per_w)], w_v)

        def gather_start(g, slot, sem):
            pltpu.async_copy(
                table_hbm.at[idx_v.at[pl.ds(g * 3 * _PCH, 3 * _PCH)]],
                rows_v.at[slot], sem)

        def gather_wait(slot, sem):
            pltpu.make_async_copy(
                table_hbm.at[pl.ds(0, 3 * _PCH)], rows_v.at[slot], sem
            ).wait()

        def out_desc(g, slot, sem):
            return pltpu.make_async_copy(
                out_v.at[slot],
                out_hbm.at[pl.ds(pbase + g * _PCH, _PCH)], sem)

        def compute(g, slot):
            def pbody(pp, carry):
                base = g * (3 * _PCH) + 3 * pp
                iv = jnp.zeros((16,), jnp.int32) + base
                w0 = plsc.load_gather(w_v, [iv])
                w1 = plsc.load_gather(w_v, [iv + 1])
                w2 = plsc.load_gather(w_v, [iv + 2])
                for cc in range(D_FEAT // 16):
                    sl = pl.ds(cc * 16, 16)
                    acc = (rows_v[slot, 3 * pp, sl] * w0
                           + rows_v[slot, 3 * pp + 1, sl] * w1
                           + rows_v[slot, 3 * pp + 2, sl] * w2)
                    out_v[slot, pp, sl] = acc
                return carry
            lax.fori_loop(0, _PCH, pbody, 0)

        gather_start(0, 0, gsem0)
        gather_start(1, 1, gsem1)

        def body(h, carry):
            g0 = 2 * h
            g1 = 2 * h + 1

            gather_wait(0, gsem0)

            @pl.when(h > 0)
            def _():
                out_desc(g0 - 2, 0, osem0).wait()
            compute(g0, 0)
            out_desc(g0, 0, osem0).start()

            @pl.when(g0 + 2 < nch)
            def _():
                gather_start(g0 + 2, 0, gsem0)

            gather_wait(1, gsem1)

            @pl.when(h > 0)
            def _():
                out_desc(g1 - 2, 1, osem1).wait()
            compute(g1, 1)
            out_desc(g1, 1, osem1).start()

            @pl.when(g1 + 2 < nch)
            def _():
                gather_start(g1 + 2, 1, gsem1)
            return carry

        lax.fori_loop(0, nch // 2, body, 0)
        out_desc(nch - 2, 0, osem0).wait()
        out_desc(nch - 1, 1, osem1).wait()

    return k(table, idx_flat, w_flat)


_B_MLP = 512
_NB = N_TO // _B_MLP


def _mlp_body(tf_ref, interpa_ref, interpb_ref,
              w0a_ref, w0b_ref, w1_ref, w2_ref,
              b0_ref, g0_ref, be0_ref, b1_ref, g1_ref, be1_ref,
              b2_ref, g2_ref, be2_ref,
              out_ref, y0_ref, y1_ref, y2_ref,
              inb_ref, inb2_ref, outb_ref, outb2_ref,
              st0_ref, st1_ref, st2_ref,
              isem, osem):
    p = pl.program_id(0)
    j = pl.program_id(1)
    s = j % 2
    last = _NB - 1

    def accum(st_ref, y):
        part = jnp.concatenate(
            [jnp.sum(y, axis=0, keepdims=True),
             jnp.sum(y * y, axis=0, keepdims=True)], axis=0)
        st_ref[...] = jnp.where(j == 0, part, st_ref[...] + part)

    def bn(y, st_ref, g_ref, be_ref):
        mean = st_ref[0:1, :] / N_TO
        var = st_ref[1:2, :] / N_TO - mean * mean
        x = g_ref[...] * (y - mean) * lax.rsqrt(var + EPS) + be_ref[...]
        return jnp.maximum(x, 0.0)

    def out_copy(buf_ref, y_ref, width, jj):
        return pltpu.make_async_copy(
            buf_ref.at[jj % 2],
            y_ref.at[pl.ds((jj % _NB) * _B_MLP, _B_MLP), pl.ds(0, width)],
            osem.at[jj % 2])

    def in_copy(y_ref, buf_ref, width, jj):
        return pltpu.make_async_copy(
            y_ref.at[pl.ds((jj % _NB) * _B_MLP, _B_MLP), pl.ds(0, width)],
            buf_ref.at[jj % 2],
            isem.at[jj % 2])

    @pl.when(p == 0)
    def _():
        interp = jnp.where(j < _NB // 2, interpa_ref[...], interpb_ref[...])
        y = (jnp.dot(tf_ref[...], w0a_ref[...],
                     preferred_element_type=jnp.float32)
             + jnp.dot(interp, w0b_ref[...],
                       preferred_element_type=jnp.float32)
             + b0_ref[...])
        accum(st0_ref, y)

        @pl.when(j >= 1)
        def _():
            out_copy(outb_ref, y0_ref, 512, j - 1).wait()
        outb_ref[s] = y
        out_copy(outb_ref, y0_ref, 512, j).start()

    @pl.when(p == 1)
    def _():
        @pl.when(j == 0)
        def _():
            out_copy(outb_ref, y0_ref, 512, last).wait()
            in_copy(y0_ref, inb_ref, 512, 0).start()
        in_copy(y0_ref, inb_ref, 512, j).wait()

        @pl.when(j < last)
        def _():
            in_copy(y0_ref, inb_ref, 512, j + 1).start()
        x = bn(inb_ref[s], st0_ref, g0_ref, be0_ref)
        y = jnp.dot(x, w1_ref[...],
                    preferred_element_type=jnp.float32) + b1_ref[...]
        accum(st1_ref, y)

        @pl.when(j >= 1)
        def _():
            out_copy(outb_ref, y1_ref, 512, j - 1).wait()
        outb_ref[s] = y
        out_copy(outb_ref, y1_ref, 512, j).start()

    @pl.when(p == 2)
    def _():
        @pl.when(j == 0)
        def _():
            out_copy(outb_ref, y1_ref, 512, last).wait()
            in_copy(y1_ref, inb_ref, 512, 0).start()
        in_copy(y1_ref, inb_ref, 512, j).wait()

        @pl.when(j < last)
        def _():
            in_copy(y1_ref, inb_ref, 512, j + 1).start()
        x = bn(inb_ref[s], st1_ref, g1_ref, be1_ref)
        y = jnp.dot(x, w2_ref[...],
                    preferred_element_type=jnp.float32) + b2_ref[...]
        accum(st2_ref, y)

        @pl.when(j >= 1)
        def _():
            out_copy(outb2_ref, y2_ref, 256, j - 1).wait()
        outb2_ref[s] = y
        out_copy(outb2_ref, y2_ref, 256, j).start()

    @pl.when(p == 3)
    def _():
        @pl.when(j == 0)
        def _():
            out_copy(outb2_ref, y2_ref, 256, last).wait()
            in_copy(y2_ref, inb2_ref, 256, 0).start()
        in_copy(y2_ref, inb2_ref, 256, j).wait()

        @pl.when(j < last)
        def _():
            in_copy(y2_ref, inb2_ref, 256, j + 1).start()
        out_ref[...] = bn(inb2_ref[s], st2_ref, g2_ref, be2_ref)


def _mlp(to_features, interpa, interpb, w0aT, w0bT, w1T, w2T,
         b0, g0, be0, b1, g1, be1, b2, g2, be2):
    def blk(jx):
        return lambda p, j: (jnp.where(p == jx, j, 0), 0)

    half = _NB // 2

    def blk_a(p, j):
        return (jnp.where((p == 0) & (j < half), j, 0), 0)

    def blk_b(p, j):
        return (jnp.where((p == 0) & (j >= half), j - half, 0), 0)

    def const2(p, j):
        return (0, 0)

    outs = pl.pallas_call(
        _mlp_body,
        grid=(4, _NB),
        in_specs=[
            pl.BlockSpec((_B_MLP, D_TO), blk(0)),
            pl.BlockSpec((_B_MLP, D_FEAT), blk_a),
            pl.BlockSpec((_B_MLP, D_FEAT), blk_b),
            pl.BlockSpec((D_TO, 512), const2),
            pl.BlockSpec((D_FEAT, 512), const2),
            pl.BlockSpec((512, 512), const2),
            pl.BlockSpec((512, 256), const2),
        ] + [pl.BlockSpec((1, 512), const2)] * 6
          + [pl.BlockSpec((1, 256), const2)] * 3,
        out_specs=[
            pl.BlockSpec((_B_MLP, 256), blk(3)),
            pl.BlockSpec(memory_space=pltpu.MemorySpace.HBM),
            pl.BlockSpec(memory_space=pltpu.MemorySpace.HBM),
            pl.BlockSpec(memory_space=pltpu.MemorySpace.HBM),
        ],
        out_shape=[
            jax.ShapeDtypeStruct((N_TO, 256), jnp.float32),
            jax.ShapeDtypeStruct((N_TO, 512), jnp.float32),
            jax.ShapeDtypeStruct((N_TO, 512), jnp.float32),
            jax.ShapeDtypeStruct((N_TO, 256), jnp.float32),
        ],
        scratch_shapes=[
            pltpu.VMEM((2, _B_MLP, 512), jnp.float32),
            pltpu.VMEM((2, _B_MLP, 256), jnp.float32),
            pltpu.VMEM((2, _B_MLP, 512), jnp.float32),
            pltpu.VMEM((2, _B_MLP, 256), jnp.float32),
            pltpu.VMEM((2, 512), jnp.float32),
            pltpu.VMEM((2, 512), jnp.float32),
            pltpu.VMEM((2, 256), jnp.float32),
            pltpu.SemaphoreType.DMA((2,)),
            pltpu.SemaphoreType.DMA((2,)),
        ],
    )(to_features, interpa, interpb, w0aT, w0bT, w1T, w2T,
      b0, g0, be0, b1, g1, be1, b2, g2, be2)
    return outs[0]


def kernel(from_coords, from_features, to_coords, to_features,
           W0, b0, g0, be0, W1, b1, g1, be1, W2, b2, g2, be2):
    fromT = from_coords.T
    h = N_TO // 2
    idxa, wa = _knn(to_coords[:h], fromT)
    interpa = _interp_sc(from_features, idxa.reshape(-1), wa.reshape(-1))
    idxb, wb = _knn(to_coords[h:], fromT)
    interpb = _interp_sc(from_features, idxb.reshape(-1), wb.reshape(-1))
    return _mlp(to_features, interpa, interpb,
                W0[:, :D_TO].T, W0[:, D_TO:].T, W1.T, W2.T,
                b0[None, :], g0[None, :], be0[None, :],
                b1[None, :], g1[None, :], be1[None, :],
                b2[None, :], g2[None, :], be2[None, :])
